# Initial kernel scaffold; baseline (speedup 1.0000x reference)
import jax
import jax.numpy as jnp
from jax.experimental import pallas as pl


def kernel(x, edge_index, edge_attr, W0, b0, Wl1, bl1, Wr1, br1, g1, be1, Wl2, bl2, Wr2, br2, g2, be2, Wf, bf):
    raise NotImplementedError("write your pallas kernel here")


# speedup vs baseline: 1.6942x; 1.6942x over previous
import functools

import jax
import jax.numpy as jnp
from jax import lax
from jax.experimental import pallas as pl
from jax.experimental.pallas import tpu as pltpu
from jax.experimental.pallas import tpu_sc as plsc

N = 10000
E = 160000
D_IN = 256
D_H = 512

NB = 1000
NGRID = N // NB
NFB = 4
FB = D_H // NFB

NT = 16
NC = 2
CH = 112
NCH = 90
EPT = NCH * CH
NH = 5120
TRASH = NH
NACC = NH + 8
N_PAD = NC * NH
RPT = NH // NT
WCH = 64
RCH = RPT // WCH
EPS = 1e-5


def _dotT(a, w):
    return lax.dot_general(a, w, (((1,), (1,)), ((), ())),
                           preferred_element_type=jnp.float32)


def _stage_a_body(x_ref, w_ref, b_ref, o_ref):
    z = _dotT(x_ref[...], w_ref[...]) + b_ref[...]
    for i in range(NFB):
        o_ref[i] = z[:, i * FB:(i + 1) * FB]


def _stage_a(x, w0, b0):
    return pl.pallas_call(
        _stage_a_body,
        grid=(NGRID,),
        in_specs=[
            pl.BlockSpec((NB, D_IN), lambda n: (n, 0)),
            pl.BlockSpec((D_H, D_IN), lambda n: (0, 0)),
            pl.BlockSpec((1, D_H), lambda n: (0, 0)),
        ],
        out_specs=pl.BlockSpec((NFB, NB, FB), lambda n: (0, n, 0)),
        out_shape=jax.ShapeDtypeStruct((NFB, N, FB), jnp.float32),
    )(x, w0, b0)


def _stage_mm_body(s_ref, c_ref, h_ref, wl_ref, bl_ref, wr_ref, br_ref,
                   z_ref, s1_ref, s2_ref):
    n = pl.program_id(0)
    cnt = jnp.maximum(c_ref[...], 1.0)
    mean = jnp.concatenate([s_ref[i] for i in range(NFB)], axis=1) / cnt
    h = jnp.concatenate([h_ref[i] for i in range(NFB)], axis=1)
    z = (_dotT(mean, wl_ref[...]) + bl_ref[...]
         + _dotT(h, wr_ref[...]) + br_ref[...])
    for i in range(NFB):
        z_ref[i] = z[:, i * FB:(i + 1) * FB]
    ps1 = jnp.sum(z, axis=0, keepdims=True)
    ps2 = jnp.sum(z * z, axis=0, keepdims=True)

    @pl.when(n == 0)
    def _():
        s1_ref[...] = ps1
        s2_ref[...] = ps2

    @pl.when(n > 0)
    def _():
        s1_ref[...] += ps1
        s2_ref[...] += ps2


def _stage_mm(summed, cnt, h4, wl, bl, wr, br):
    return pl.pallas_call(
        _stage_mm_body,
        grid=(NGRID,),
        in_specs=[
            pl.BlockSpec((NFB, NB, FB), lambda n: (0, n, 0)),
            pl.BlockSpec((NB, 1), lambda n: (n, 0)),
            pl.BlockSpec((NFB, NB, FB), lambda n: (0, n, 0)),
            pl.BlockSpec((D_H, D_H), lambda n: (0, 0)),
            pl.BlockSpec((1, D_H), lambda n: (0, 0)),
            pl.BlockSpec((D_H, D_H), lambda n: (0, 0)),
            pl.BlockSpec((1, D_H), lambda n: (0, 0)),
        ],
        out_specs=[
            pl.BlockSpec((NFB, NB, FB), lambda n: (0, n, 0)),
            pl.BlockSpec((1, D_H), lambda n: (0, 0)),
            pl.BlockSpec((1, D_H), lambda n: (0, 0)),
        ],
        out_shape=[
            jax.ShapeDtypeStruct((NFB, N, FB), jnp.float32),
            jax.ShapeDtypeStruct((1, D_H), jnp.float32),
            jax.ShapeDtypeStruct((1, D_H), jnp.float32),
        ],
    )(summed, cnt, h4, wl, bl, wr, br)


def _stage_bn_body(z_ref, s1_ref, s2_ref, g_ref, be_ref, o_ref):
    m = s1_ref[...] * (1.0 / N)
    v = s2_ref[...] * (1.0 / N) - m * m
    inv = lax.rsqrt(v + EPS)
    a = g_ref[...] * inv
    b = be_ref[...] - m * a
    for i in range(NFB):
        sl = slice(i * FB, (i + 1) * FB)
        o_ref[i] = jnp.maximum(z_ref[i] * a[:, sl] + b[:, sl], 0.0)


def _stage_bn(z4, s1, s2, g, be):
    return pl.pallas_call(
        _stage_bn_body,
        grid=(NGRID,),
        in_specs=[
            pl.BlockSpec((NFB, NB, FB), lambda n: (0, n, 0)),
            pl.BlockSpec((1, D_H), lambda n: (0, 0)),
            pl.BlockSpec((1, D_H), lambda n: (0, 0)),
            pl.BlockSpec((1, D_H), lambda n: (0, 0)),
            pl.BlockSpec((1, D_H), lambda n: (0, 0)),
        ],
        out_specs=pl.BlockSpec((NFB, NB, FB), lambda n: (0, n, 0)),
        out_shape=jax.ShapeDtypeStruct((NFB, N, FB), jnp.float32),
    )(z4, s1, s2, g, be)


def _stage_out_body(z_ref, s1_ref, s2_ref, g_ref, be_ref, h1_ref,
                    wf_ref, bf_ref, o_ref):
    m = s1_ref[...] * (1.0 / N)
    v = s2_ref[...] * (1.0 / N) - m * m
    inv = lax.rsqrt(v + EPS)
    a = g_ref[...] * inv
    b = be_ref[...] - m * a
    acc = jnp.zeros((NB, 1), jnp.float32)
    for i in range(NFB):
        sl = slice(i * FB, (i + 1) * FB)
        h2 = jnp.maximum(z_ref[i] * a[:, sl] + b[:, sl], 0.0)
        h = h1_ref[i] + h2
        acc = acc + jnp.sum(h * wf_ref[:, sl], axis=1, keepdims=True)
    o_ref[...] = acc + bf_ref[...]


def _stage_out(z4, s1, s2, g, be, h14, wf, bf):
    return pl.pallas_call(
        _stage_out_body,
        grid=(NGRID,),
        in_specs=[
            pl.BlockSpec((NFB, NB, FB), lambda n: (0, n, 0)),
            pl.BlockSpec((1, D_H), lambda n: (0, 0)),
            pl.BlockSpec((1, D_H), lambda n: (0, 0)),
            pl.BlockSpec((1, D_H), lambda n: (0, 0)),
            pl.BlockSpec((1, D_H), lambda n: (0, 0)),
            pl.BlockSpec((NFB, NB, FB), lambda n: (0, n, 0)),
            pl.BlockSpec((1, D_H), lambda n: (0, 0)),
            pl.BlockSpec((1, 1), lambda n: (0, 0)),
        ],
        out_specs=pl.BlockSpec((NB, 1), lambda n: (n, 0)),
        out_shape=jax.ShapeDtypeStruct((N, 1), jnp.float32),
    )(z4, s1, s2, g, be, h14, wf, bf)


def _sc_agg_body(with_count, h4, src3, dst3, zeros_h, zeros1_h,
                 ones_h, *refs):
    if with_count:
        (out4, cnt_out, sidx, didx, gbuf, zwbuf, wbuf, acc,
         cstage, onesb, cacc) = refs
    else:
        (out4, sidx, didx, gbuf, zwbuf, wbuf, acc) = refs

    c = lax.axis_index("c")
    s = lax.axis_index("s")

    pltpu.sync_copy(src3.at[s], sidx)
    pltpu.sync_copy(dst3.at[s], didx)
    pltpu.sync_copy(zeros_h, zwbuf)

    off_v = jnp.full((16,), c * NH, jnp.int32)
    nhu_v = jnp.full((16,), NH, jnp.uint32)

    def remap(j, carry):
        for k in range(CH // 16):
            dv = didx[j, pl.ds(k * 16, 16)]
            tu = plsc.bitcast(dv - off_v, jnp.uint32)
            didx[j, pl.ds(k * 16, 16)] = plsc.bitcast(
                jnp.minimum(tu, nhu_v), jnp.int32)
        return carry
    lax.fori_loop(0, NCH, remap, 0)

    if with_count:
        pltpu.sync_copy(zeros1_h, cstage)
        pltpu.sync_copy(ones_h, onesb)
        pltpu.sync_copy(cstage.at[pl.ds(0, RPT)],
                        cacc.at[pl.ds(s * RPT, RPT)])

        @pl.when(s == NT - 1)
        def _():
            pltpu.sync_copy(cstage.at[pl.ds(0, 8)],
                            cacc.at[pl.ds(NH, 8)])
        plsc.subcore_barrier()

        def cscat(j, carry):
            pltpu.sync_copy(onesb.at[pl.ds(0, CH)],
                            cacc.at[didx.at[j]], add=True)
            return carry
        lax.fori_loop(0, NCH, cscat, 0)
        plsc.subcore_barrier()

        pltpu.sync_copy(cacc.at[pl.ds(s * RPT, RPT)],
                        cstage.at[pl.ds(0, RPT)])
        pltpu.sync_copy(cstage.at[pl.ds(0, RPT)],
                        cnt_out.at[pl.ds(c * NH + s * RPT, RPT)])

    for fb in range(NFB):
        hsrc = h4.at[fb]
        osrc = out4.at[fb]

        plsc.subcore_barrier()
        for k in range(RCH):
            pltpu.sync_copy(zwbuf, acc.at[pl.ds(s * RPT + k * WCH, WCH)])

        @pl.when(s == NT - 1)
        def _():
            pltpu.sync_copy(zwbuf.at[pl.ds(0, 8)], acc.at[pl.ds(NH, 8)])
        plsc.subcore_barrier()

        def chunk(j, carry):
            pltpu.sync_copy(hsrc.at[sidx.at[j]], gbuf)
            pltpu.sync_copy(gbuf, acc.at[didx.at[j]], add=True)
            return carry
        lax.fori_loop(0, NCH, chunk, 0)

        plsc.subcore_barrier()
        for k in range(RCH):
            r0 = s * RPT + k * WCH
            pltpu.sync_copy(acc.at[pl.ds(r0, WCH)], wbuf)
            pltpu.sync_copy(wbuf, osrc.at[pl.ds(c * NH + r0, WCH)])


def _sc_agg(h4, src3, dst3, zeros_h, zeros1_h, ones_h, with_count):
    mesh = plsc.VectorSubcoreMesh(core_axis_name="c", subcore_axis_name="s")
    out_type = [jax.ShapeDtypeStruct((NFB, N_PAD, FB), jnp.float32)]
    scratch = [
        pltpu.VMEM((NCH, CH), jnp.int32),
        pltpu.VMEM((NCH, CH), jnp.int32),
        pltpu.VMEM((CH, FB), jnp.float32),
        pltpu.VMEM((WCH, FB), jnp.float32),
        pltpu.VMEM((WCH, FB), jnp.float32),
        pltpu.VMEM_SHARED((NACC, FB), jnp.float32),
    ]
    if with_count:
        out_type.append(jax.ShapeDtypeStruct((N_PAD,), jnp.float32))
        scratch += [
            pltpu.VMEM((RPT,), jnp.float32),
            pltpu.VMEM((128,), jnp.float32),
            pltpu.VMEM_SHARED((NACC,), jnp.float32),
        ]
    fn = pl.kernel(
        functools.partial(_sc_agg_body, with_count),
        out_type=out_type,
        mesh=mesh,
        scratch_types=scratch,
    )
    res = fn(h4, src3, dst3, zeros_h, zeros1_h, ones_h)
    return res if with_count else res[0]


@jax.jit
def kernel(x, edge_index, edge_attr, W0, b0, Wl1, bl1, Wr1, br1, g1, be1,
           Wl2, bl2, Wr2, br2, g2, be2, Wf, bf):
    del edge_attr
    src = edge_index[0].astype(jnp.int32)
    dst = edge_index[1].astype(jnp.int32)
    pad = NT * EPT - E
    src3 = jnp.concatenate(
        [src, jnp.zeros((pad,), jnp.int32)]).reshape(NT, NCH, CH)
    dst3 = jnp.concatenate(
        [dst, jnp.full((pad,), 1 << 29, jnp.int32)]).reshape(NT, NCH, CH)
    zeros_h = jnp.zeros((WCH, FB), jnp.float32)
    zeros1_h = jnp.zeros((RPT,), jnp.float32)
    ones_h = jnp.ones((128,), jnp.float32)

    b0r = b0.reshape(1, D_H)
    bl1r = bl1.reshape(1, D_H)
    br1r = br1.reshape(1, D_H)
    g1r = g1.reshape(1, D_H)
    be1r = be1.reshape(1, D_H)
    bl2r = bl2.reshape(1, D_H)
    br2r = br2.reshape(1, D_H)
    g2r = g2.reshape(1, D_H)
    be2r = be2.reshape(1, D_H)
    bfr = bf.reshape(1, 1)

    h0 = _stage_a(x, W0, b0r)
    summed1, cnt = _sc_agg(h0, src3, dst3, zeros_h, zeros1_h,
                           ones_h, with_count=True)
    cnt2 = cnt.reshape(N_PAD, 1)
    z1, s11, s12 = _stage_mm(summed1, cnt2, h0, Wl1, bl1r, Wr1, br1r)
    h1 = _stage_bn(z1, s11, s12, g1r, be1r)
    summed2 = _sc_agg(h1, src3, dst3, zeros_h, zeros1_h,
                      ones_h, with_count=False)
    z2, s21, s22 = _stage_mm(summed2, cnt2, h1, Wl2, bl2r, Wr2, br2r)
    out = _stage_out(z2, s21, s22, g2r, be2r, h1, Wf, bfr)
    return out
